# baseline (speedup 1.0000x reference)
import functools

import jax
import jax.numpy as jnp
from jax import lax
from jax.experimental import pallas as pl
from jax.experimental.pallas import tpu as pltpu
from jax.experimental.pallas import tpu_sc as plsc

_B = 16384
_D = 64
_NC = 2
_NS = 16
_NW = _NC * _NS
_BPW = _B // _NW


def _gather_body(uid_hbm, iid_hbm, gu_t, gi_t, mu_t, mi_t,
                 gu_o, gi_o, mu_o, mi_o,
                 uidx, iidx, buf0, buf1, sem0, sem1, wsem):
    wid = lax.axis_index("s") * _NC + lax.axis_index("c")
    base = wid * _BPW
    pltpu.sync_copy(uid_hbm.at[pl.ds(base, _BPW)], uidx)
    pltpu.sync_copy(iid_hbm.at[pl.ds(base, _BPW)], iidx)

    plan = ((gu_t, uidx, gu_o, buf0, sem0),
            (gi_t, iidx, gi_o, buf1, sem1),
            (mu_t, uidx, mu_o, buf0, sem0),
            (mi_t, iidx, mi_o, buf1, sem1))

    handles = [None, None, None, None]
    writes = [None, None, None, None]
    for k, (tab, idx, _, buf, sem) in enumerate(plan[:2]):
        handles[k] = pltpu.async_copy(tab.at[idx], buf, sem)
    for k, (tab, idx, out, buf, sem) in enumerate(plan):
        handles[k].wait()
        writes[k] = pltpu.async_copy(buf, out.at[pl.ds(base, _BPW)], wsem)
        if k + 2 < 4:
            nt, nidx, _, nbuf, nsem = plan[k + 2]
            writes[k].wait()
            handles[k + 2] = pltpu.async_copy(nt.at[nidx], nbuf, nsem)
    writes[2].wait()
    writes[3].wait()


@functools.cache
def _gather4():
    return pl.kernel(
        _gather_body,
        out_type=[jax.ShapeDtypeStruct((_B, _D), jnp.float32)] * 4,
        mesh=plsc.VectorSubcoreMesh(core_axis_name="c", subcore_axis_name="s"),
        compiler_params=pltpu.CompilerParams(use_tc_tiling_on_sc=False),
        scratch_types=[
            pltpu.VMEM((_BPW,), jnp.int32),
            pltpu.VMEM((_BPW,), jnp.int32),
            pltpu.VMEM((_BPW, _D), jnp.float32),
            pltpu.VMEM((_BPW, _D), jnp.float32),
            pltpu.SemaphoreType.DMA,
            pltpu.SemaphoreType.DMA,
            pltpu.SemaphoreType.DMA,
        ],
    )


def _bn(h, gamma, beta):
    mean = jnp.mean(h, axis=0, keepdims=True)
    var = jnp.mean((h - mean) ** 2, axis=0, keepdims=True)
    return (h - mean) * jax.lax.rsqrt(var + 1e-5) * gamma + beta


def _mlp_body(gu, gi, mu, mi, w1, w2, w3, vecs, out):
    v = vecs[...]
    b1, g1, be1 = v[0:1], v[1:2], v[2:3]
    b2, g2, be2 = v[3:4, :64], v[4:5, :64], v[5:6, :64]
    b3, g3, be3 = v[6:7, :32], v[6:7, 32:64], v[6:7, 64:96]
    wg, wh, pb = v[7:8, :64], v[7:8, 64:96], v[6:7, 96]

    h = (jnp.dot(mu[...], w1[0:_D], preferred_element_type=jnp.float32)
         + jnp.dot(mi[...], w1[_D:2 * _D], preferred_element_type=jnp.float32)
         + b1)
    h = _bn(jnp.maximum(h, 0.0), g1, be1)
    h = jnp.dot(h, w2[...], preferred_element_type=jnp.float32) + b2
    h = _bn(jnp.maximum(h, 0.0), g2, be2)
    h = jnp.dot(h, w3[...], preferred_element_type=jnp.float32) + b3
    h = _bn(jnp.maximum(h, 0.0), g3, be3)

    gmf = gu[...] * gi[...]
    pred = (jnp.sum(gmf * wg, axis=1) + jnp.sum(h * wh, axis=1) + pb)
    out[...] = pred


def kernel(user_ids, item_ids, params):
    gu, gi, mu, mi = _gather4()(
        user_ids, item_ids,
        params['gmf_user'], params['gmf_item'],
        params['mlp_user'], params['mlp_item'])

    (w1, b1, g1, be1), (w2, b2, g2, be2), (w3, b3, g3, be3) = params['mlp']
    pw = params['pred_W'][:, 0]
    z128 = jnp.zeros((128,), jnp.float32)
    row6 = jnp.concatenate([b3, g3, be3, params['pred_b'], jnp.zeros((31,), jnp.float32)])
    row7 = jnp.concatenate([pw, jnp.zeros((32,), jnp.float32)])
    vecs = jnp.stack([
        b1, g1, be1,
        jnp.concatenate([b2, z128[:64]]),
        jnp.concatenate([g2, z128[:64]]),
        jnp.concatenate([be2, z128[:64]]),
        row6, row7,
    ])

    return pl.pallas_call(
        _mlp_body,
        out_shape=jax.ShapeDtypeStruct((_B,), jnp.float32),
        compiler_params=pltpu.CompilerParams(vmem_limit_bytes=100 * 1024 * 1024),
    )(gu, gi, mu, mi, w1, w2, w3, vecs)
